# Initial kernel scaffold; baseline (speedup 1.0000x reference)
import functools

import jax
import jax.numpy as jnp
from jax import lax
from jax.experimental import pallas as pl
from jax.experimental.pallas import tpu as pltpu
from jax.experimental.pallas import tpu_sc as plsc

NC = 2
NS = 16
CH = 128


def _make_deg(n_pad, e_pad):
    nw = NC * NS
    ept = e_pad // nw
    n_chunks = ept // CH
    rpt = n_pad // NS
    mesh = plsc.VectorSubcoreMesh(core_axis_name="c", subcore_axis_name="s")

    @functools.partial(
        pl.kernel,
        mesh=mesh,
        out_type=jax.ShapeDtypeStruct((NC, n_pad, 16), jnp.float32),
        scratch_types=[
            pltpu.VMEM((CH,), jnp.int32),
            pltpu.VMEM((CH, 16), jnp.float32),
            pltpu.VMEM_SHARED((n_pad, 16), jnp.float32),
        ],
    )
    def deg(dst_hbm, out_hbm, didx, vals, acc):
        c = lax.axis_index("c")
        s = lax.axis_index("s")
        wid = s * NC + c

        def fill(v):
            def body(i, _):
                vals[i, pl.ds(0, 16)] = jnp.full((16,), v, jnp.float32)
                return 0
            lax.fori_loop(0, CH, body, 0)

        fill(0.0)
        row0 = s * rpt

        def zcopy(k, _):
            pltpu.sync_copy(vals, acc.at[pl.ds(row0 + k * CH, CH), :])
            return 0

        lax.fori_loop(0, rpt // CH, zcopy, 0)
        fill(1.0)
        plsc.subcore_barrier()

        ebase = wid * ept

        def step(i, _):
            base = pl.multiple_of(ebase + i * CH, CH)
            pltpu.sync_copy(dst_hbm.at[pl.ds(base, CH)], didx)
            pltpu.sync_copy(vals, acc.at[didx], add=True)
            return 0

        lax.fori_loop(0, n_chunks, step, 0)
        plsc.subcore_barrier()

        def ocopy(k, _):
            r = pl.multiple_of(row0 + k * CH, CH)
            pltpu.sync_copy(acc.at[pl.ds(r, CH), :], vals)
            pltpu.sync_copy(vals, out_hbm.at[c, pl.ds(r, CH), :])
            return 0

        lax.fori_loop(0, rpt // CH, ocopy, 0)

    return deg


def _make_agg(n_pad, e_pad, d):
    nw = NC * NS
    ept = e_pad // nw
    n_chunks = ept // CH
    rpt = n_pad // NS
    mesh = plsc.VectorSubcoreMesh(core_axis_name="c", subcore_axis_name="s")

    @functools.partial(
        pl.kernel,
        mesh=mesh,
        out_type=jax.ShapeDtypeStruct((NC, n_pad, d), jnp.float32),
        scratch_types=[
            pltpu.VMEM((CH,), jnp.int32),
            pltpu.VMEM((CH,), jnp.int32),
            pltpu.VMEM((CH, d), jnp.float32),
            pltpu.VMEM_SHARED((n_pad, d), jnp.float32),
            pltpu.SemaphoreType.DMA,
        ],
    )
    def agg(y_hbm, src_hbm, dst_hbm, out_hbm, sidx, didx, rows, acc, sem):
        c = lax.axis_index("c")
        s = lax.axis_index("s")
        wid = s * NC + c

        def zbuf(i, _):
            rows[i // 8, pl.ds((i % 8) * 16, 16)] = jnp.zeros((16,), jnp.float32)
            return 0

        lax.fori_loop(0, CH * d // 16, zbuf, 0)
        row0 = s * rpt

        def zcopy(k, _):
            pltpu.sync_copy(rows, acc.at[pl.ds(row0 + k * CH, CH), :])
            return 0

        lax.fori_loop(0, rpt // CH, zcopy, 0)
        plsc.subcore_barrier()

        ebase = wid * ept

        def step(i, _):
            base = pl.multiple_of(ebase + i * CH, CH)
            pltpu.sync_copy(src_hbm.at[pl.ds(base, CH)], sidx)
            pltpu.sync_copy(dst_hbm.at[pl.ds(base, CH)], didx)
            pltpu.async_copy(y_hbm.at[sidx], rows, sem).wait()
            pltpu.sync_copy(rows, acc.at[didx], add=True)
            return 0

        lax.fori_loop(0, n_chunks, step, 0)
        plsc.subcore_barrier()

        def ocopy(k, _):
            r = pl.multiple_of(row0 + k * CH, CH)
            pltpu.sync_copy(acc.at[pl.ds(r, CH), :], rows)
            pltpu.sync_copy(rows, out_hbm.at[c, pl.ds(r, CH), :])
            return 0

        lax.fori_loop(0, rpt // CH, ocopy, 0)

    return agg


def _tc1_body(x_ref, w_ref, degp_ref, y_ref, dinv_ref):
    deg = 1.0 + degp_ref[0, :, 0] + degp_ref[1, :, 0]
    dinv = lax.rsqrt(deg)
    xw = jnp.dot(x_ref[...], w_ref[...], preferred_element_type=jnp.float32)
    y_ref[...] = xw * dinv[:, None]
    dinv_ref[...] = dinv[:, None]


def _tc2_body(aggp_ref, y1_ref, dinv_ref, b1_ref, t_ref, stats_ref, acc_ref):
    i = pl.program_id(0)
    t = dinv_ref[...] * (aggp_ref[0] + aggp_ref[1] + y1_ref[...]) + b1_ref[...]
    t_ref[...] = t
    ps = jnp.sum(t, axis=0, keepdims=True)
    pq = jnp.sum(t * t, axis=0, keepdims=True)

    @pl.when(i == 0)
    def _():
        acc_ref[0:1, :] = ps
        acc_ref[1:2, :] = pq

    @pl.when(i > 0)
    def _():
        acc_ref[0:1, :] += ps
        acc_ref[1:2, :] += pq

    @pl.when(i == pl.num_programs(0) - 1)
    def _():
        stats_ref[...] = acc_ref[...]


def _tc3_body(n, t_ref, stats_ref, g_ref, be_ref, dinv_ref, w_ref, y2_ref):
    mean = stats_ref[0:1, :] / n
    var = stats_ref[1:2, :] / n - mean * mean
    inv = lax.rsqrt(var + 1e-5)
    h = (t_ref[...] - mean) * inv * g_ref[...] + be_ref[...]
    h = jnp.maximum(h, 0.0)
    y2_ref[...] = dinv_ref[...] * jnp.dot(
        h, w_ref[...], preferred_element_type=jnp.float32)


def _tc4_body(aggp_ref, y2_ref, dinv_ref, b2_ref, out_ref):
    out_ref[...] = (
        dinv_ref[...] * (aggp_ref[0] + aggp_ref[1] + y2_ref[...]) + b2_ref[...])


def _row_specs(r, d):
    return pl.BlockSpec((r, d), lambda i: (i, 0))


def kernel(x, edge_index, W1, b1, gamma, beta, W2, b2):
    n, d_in = x.shape
    d_h = W1.shape[1]
    d_out = W2.shape[1]
    e = edge_index.shape[1]

    grain = NC * NS * CH
    e_pad = -(-e // grain) * grain
    n_pad = -(-n // (NS * CH)) * (NS * CH)

    src = edge_index[0].astype(jnp.int32)
    dst = edge_index[1].astype(jnp.int32)
    src_p = jnp.concatenate([src, jnp.zeros((e_pad - e,), jnp.int32)])
    dst_p = jnp.concatenate([dst, jnp.full((e_pad - e,), n, jnp.int32)])

    degp = _make_deg(n_pad, e_pad)(dst_p)

    r = 2000
    grid = n // r
    f32 = jnp.float32

    y1, dinv = pl.pallas_call(
        _tc1_body,
        grid=(grid,),
        in_specs=[
            _row_specs(r, d_in),
            pl.BlockSpec((d_in, d_h), lambda i: (0, 0)),
            pl.BlockSpec((2, r, 16), lambda i: (0, i, 0)),
        ],
        out_specs=[_row_specs(r, d_h), pl.BlockSpec((r, 1), lambda i: (i, 0))],
        out_shape=[
            jax.ShapeDtypeStruct((n, d_h), f32),
            jax.ShapeDtypeStruct((n, 1), f32),
        ],
    )(x, W1, degp)

    agg_fn = _make_agg(n_pad, e_pad, d_h)
    agg1 = agg_fn(y1, src_p, dst_p)

    t, stats = pl.pallas_call(
        _tc2_body,
        grid=(grid,),
        in_specs=[
            pl.BlockSpec((2, r, d_h), lambda i: (0, i, 0)),
            _row_specs(r, d_h),
            pl.BlockSpec((r, 1), lambda i: (i, 0)),
            pl.BlockSpec((d_h,), lambda i: (0,)),
        ],
        out_specs=[_row_specs(r, d_h), pl.BlockSpec((8, d_h), lambda i: (0, 0))],
        out_shape=[
            jax.ShapeDtypeStruct((n, d_h), f32),
            jax.ShapeDtypeStruct((8, d_h), f32),
        ],
        scratch_shapes=[pltpu.VMEM((8, d_h), f32)],
    )(agg1, y1, dinv, b1)

    y2 = pl.pallas_call(
        functools.partial(_tc3_body, float(n)),
        grid=(grid,),
        in_specs=[
            _row_specs(r, d_h),
            pl.BlockSpec((8, d_h), lambda i: (0, 0)),
            pl.BlockSpec((d_h,), lambda i: (0,)),
            pl.BlockSpec((d_h,), lambda i: (0,)),
            pl.BlockSpec((r, 1), lambda i: (i, 0)),
            pl.BlockSpec((d_h, d_out), lambda i: (0, 0)),
        ],
        out_specs=_row_specs(r, d_out),
        out_shape=jax.ShapeDtypeStruct((n, d_out), f32),
    )(t, stats, gamma, beta, dinv, W2)

    agg2 = agg_fn(y2, src_p, dst_p)

    out = pl.pallas_call(
        _tc4_body,
        grid=(grid,),
        in_specs=[
            pl.BlockSpec((2, r, d_out), lambda i: (0, i, 0)),
            _row_specs(r, d_out),
            pl.BlockSpec((r, 1), lambda i: (i, 0)),
            pl.BlockSpec((d_out,), lambda i: (0,)),
        ],
        out_specs=_row_specs(r, d_out),
        out_shape=jax.ShapeDtypeStruct((n, d_out), f32),
    )(agg2, y2, dinv, b2)

    return out


# speedup vs baseline: 7.6279x; 7.6279x over previous
import functools

import jax
import jax.numpy as jnp
from jax import lax
from jax.experimental import pallas as pl
from jax.experimental.pallas import tpu as pltpu
from jax.experimental.pallas import tpu_sc as plsc

NC = 2
NS = 16
CH = 128
EB = 2048


def _make_deg(n_pad, e_pad):
    nw = NC * NS
    ept = e_pad // nw
    assert ept % EB == 0
    mesh = plsc.VectorSubcoreMesh(core_axis_name="c", subcore_axis_name="s",
                                  num_cores=NC, num_subcores=NS)

    @functools.partial(
        pl.kernel,
        mesh=mesh,
        out_type=jax.ShapeDtypeStruct((nw, n_pad), jnp.float32),
        scratch_types=[
            pltpu.VMEM((EB,), jnp.int32),
            pltpu.VMEM((n_pad,), jnp.float32),
        ],
        compiler_params=pltpu.CompilerParams(needs_layout_passes=False),
    )
    def deg(dst_hbm, out_hbm, didx, cnt):
        c = lax.axis_index("c")
        s = lax.axis_index("s")
        wid = s * NC + c

        def z(i, _):
            cnt[pl.ds(i * 16, 16)] = jnp.zeros((16,), jnp.float32)
            return 0

        lax.fori_loop(0, n_pad // 16, z, 0)
        ones16 = jnp.ones((16,), jnp.float32)
        ebase = wid * ept

        def chunk(k, _):
            base = pl.multiple_of(ebase + k * EB, EB)
            pltpu.sync_copy(dst_hbm.at[pl.ds(base, EB)], didx)

            def inner(j, _):
                idx = didx[pl.ds(j * 16, 16)]
                plsc.addupdate_scatter(cnt, [idx], ones16)
                return 0

            lax.fori_loop(0, EB // 16, inner, 0)
            return 0

        lax.fori_loop(0, ept // EB, chunk, 0)
        pltpu.sync_copy(cnt, out_hbm.at[wid])

    return deg


def _make_agg(n_pad, e_pad, d):
    nw = NC * NS
    ept = e_pad // nw
    n_chunks = ept // CH
    rpt = n_pad // NS
    mesh = plsc.VectorSubcoreMesh(core_axis_name="c", subcore_axis_name="s", num_cores=NC, num_subcores=NS)

    @functools.partial(
        pl.kernel,
        mesh=mesh,
        out_type=jax.ShapeDtypeStruct((NC, n_pad, d), jnp.float32),
        scratch_types=[
            pltpu.VMEM((CH,), jnp.int32),
            pltpu.VMEM((CH,), jnp.int32),
            pltpu.VMEM((CH, d), jnp.float32),
            pltpu.VMEM_SHARED((n_pad, d), jnp.float32),
            pltpu.SemaphoreType.DMA,
        ],
    )
    def agg(y_hbm, src_hbm, dst_hbm, out_hbm, sidx, didx, rows, acc, sem):
        c = lax.axis_index("c")
        s = lax.axis_index("s")
        wid = s * NC + c

        def zbuf(i, _):
            rows[i // 8, pl.ds((i % 8) * 16, 16)] = jnp.zeros((16,), jnp.float32)
            return 0

        lax.fori_loop(0, CH * d // 16, zbuf, 0)
        row0 = s * rpt

        def zcopy(k, _):
            pltpu.sync_copy(rows, acc.at[pl.ds(row0 + k * CH, CH), :])
            return 0

        lax.fori_loop(0, rpt // CH, zcopy, 0)
        plsc.subcore_barrier()

        ebase = wid * ept

        def step(i, _):
            base = pl.multiple_of(ebase + i * CH, CH)
            pltpu.sync_copy(src_hbm.at[pl.ds(base, CH)], sidx)
            pltpu.sync_copy(dst_hbm.at[pl.ds(base, CH)], didx)
            pltpu.async_copy(y_hbm.at[sidx], rows, sem).wait()
            pltpu.sync_copy(rows, acc.at[didx], add=True)
            return 0

        lax.fori_loop(0, n_chunks, step, 0)
        plsc.subcore_barrier()

        def ocopy(k, _):
            r = pl.multiple_of(row0 + k * CH, CH)
            pltpu.sync_copy(acc.at[pl.ds(r, CH), :], rows)
            pltpu.sync_copy(rows, out_hbm.at[c, pl.ds(r, CH), :])
            return 0

        lax.fori_loop(0, rpt // CH, ocopy, 0)

    return agg


def _tc0_body(degp_ref, dinv_ref):
    deg = 1.0 + jnp.sum(degp_ref[...], axis=0)
    dinv_ref[...] = lax.rsqrt(deg)[:, None]


def _tc1_body(x_ref, w_ref, dinv_ref, y_ref):
    xw = jnp.dot(x_ref[...], w_ref[...], preferred_element_type=jnp.float32)
    y_ref[...] = xw * dinv_ref[...]


def _tc2_body(aggp_ref, y1_ref, dinv_ref, b1_ref, t_ref, stats_ref, acc_ref):
    i = pl.program_id(0)
    t = dinv_ref[...] * (aggp_ref[0] + aggp_ref[1] + y1_ref[...]) + b1_ref[...]
    t_ref[...] = t
    ps = jnp.sum(t, axis=0, keepdims=True)
    pq = jnp.sum(t * t, axis=0, keepdims=True)

    @pl.when(i == 0)
    def _():
        acc_ref[0:1, :] = ps
        acc_ref[1:2, :] = pq

    @pl.when(i > 0)
    def _():
        acc_ref[0:1, :] += ps
        acc_ref[1:2, :] += pq

    @pl.when(i == pl.num_programs(0) - 1)
    def _():
        stats_ref[...] = acc_ref[...]


def _tc3_body(n, t_ref, stats_ref, g_ref, be_ref, dinv_ref, w_ref, y2_ref):
    mean = stats_ref[0:1, :] / n
    var = stats_ref[1:2, :] / n - mean * mean
    inv = lax.rsqrt(var + 1e-5)
    h = (t_ref[...] - mean) * inv * g_ref[...] + be_ref[...]
    h = jnp.maximum(h, 0.0)
    y2_ref[...] = dinv_ref[...] * jnp.dot(
        h, w_ref[...], preferred_element_type=jnp.float32)


def _tc4_body(aggp_ref, y2_ref, dinv_ref, b2_ref, out_ref):
    out_ref[...] = (
        dinv_ref[...] * (aggp_ref[0] + aggp_ref[1] + y2_ref[...]) + b2_ref[...])


def _row_specs(r, d):
    return pl.BlockSpec((r, d), lambda i: (i, 0))


def kernel(x, edge_index, W1, b1, gamma, beta, W2, b2):
    n, d_in = x.shape
    d_h = W1.shape[1]
    d_out = W2.shape[1]
    e = edge_index.shape[1]

    grain = NC * NS * EB
    e_pad = -(-e // grain) * grain
    n_pad = -(-n // (NS * CH)) * (NS * CH)

    src = edge_index[0].astype(jnp.int32)
    dst = edge_index[1].astype(jnp.int32)
    src_p = jnp.concatenate([src, jnp.zeros((e_pad - e,), jnp.int32)])
    dst_p = jnp.concatenate([dst, jnp.full((e_pad - e,), n, jnp.int32)])

    degp = _make_deg(n_pad, e_pad)(dst_p)

    r = 2000
    grid = n // r
    f32 = jnp.float32

    dinv = pl.pallas_call(
        _tc0_body,
        in_specs=[pl.BlockSpec((NC * NS, n_pad), lambda: (0, 0))],
        out_specs=pl.BlockSpec((n_pad, 1), lambda: (0, 0)),
        out_shape=jax.ShapeDtypeStruct((n_pad, 1), f32),
    )(degp)

    y1 = pl.pallas_call(
        _tc1_body,
        grid=(grid,),
        in_specs=[
            _row_specs(r, d_in),
            pl.BlockSpec((d_in, d_h), lambda i: (0, 0)),
            pl.BlockSpec((r, 1), lambda i: (i, 0)),
        ],
        out_specs=_row_specs(r, d_h),
        out_shape=jax.ShapeDtypeStruct((n, d_h), f32),
    )(x, W1, dinv)

    agg_fn = _make_agg(n_pad, e_pad, d_h)
    agg1 = agg_fn(y1, src_p, dst_p)

    t, stats = pl.pallas_call(
        _tc2_body,
        grid=(grid,),
        in_specs=[
            pl.BlockSpec((2, r, d_h), lambda i: (0, i, 0)),
            _row_specs(r, d_h),
            pl.BlockSpec((r, 1), lambda i: (i, 0)),
            pl.BlockSpec((d_h,), lambda i: (0,)),
        ],
        out_specs=[_row_specs(r, d_h), pl.BlockSpec((8, d_h), lambda i: (0, 0))],
        out_shape=[
            jax.ShapeDtypeStruct((n, d_h), f32),
            jax.ShapeDtypeStruct((8, d_h), f32),
        ],
        scratch_shapes=[pltpu.VMEM((8, d_h), f32)],
    )(agg1, y1, dinv, b1)

    y2 = pl.pallas_call(
        functools.partial(_tc3_body, float(n)),
        grid=(grid,),
        in_specs=[
            _row_specs(r, d_h),
            pl.BlockSpec((8, d_h), lambda i: (0, 0)),
            pl.BlockSpec((d_h,), lambda i: (0,)),
            pl.BlockSpec((d_h,), lambda i: (0,)),
            pl.BlockSpec((r, 1), lambda i: (i, 0)),
            pl.BlockSpec((d_h, d_out), lambda i: (0, 0)),
        ],
        out_specs=_row_specs(r, d_out),
        out_shape=jax.ShapeDtypeStruct((n, d_out), f32),
    )(t, stats, gamma, beta, dinv, W2)

    agg2 = agg_fn(y2, src_p, dst_p)

    out = pl.pallas_call(
        _tc4_body,
        grid=(grid,),
        in_specs=[
            pl.BlockSpec((2, r, d_out), lambda i: (0, i, 0)),
            _row_specs(r, d_out),
            pl.BlockSpec((r, 1), lambda i: (i, 0)),
            pl.BlockSpec((d_out,), lambda i: (0,)),
        ],
        out_specs=_row_specs(r, d_out),
        out_shape=jax.ShapeDtypeStruct((n, d_out), f32),
    )(agg2, y2, dinv, b2)

    return out


# speedup vs baseline: 8.4849x; 1.1124x over previous
import functools

import jax
import jax.numpy as jnp
from jax import lax
from jax.experimental import pallas as pl
from jax.experimental.pallas import tpu as pltpu
from jax.experimental.pallas import tpu_sc as plsc

NC = 2
NS = 16
CH = 128
EB = 2048


def _make_deg(n_pad, e_pad):
    nw = NC * NS
    ept = e_pad // nw
    assert ept % EB == 0
    mesh = plsc.VectorSubcoreMesh(core_axis_name="c", subcore_axis_name="s",
                                  num_cores=NC, num_subcores=NS)

    @functools.partial(
        pl.kernel,
        mesh=mesh,
        out_type=jax.ShapeDtypeStruct((nw, n_pad), jnp.float32),
        scratch_types=[
            pltpu.VMEM((EB,), jnp.int32),
            pltpu.VMEM((n_pad,), jnp.float32),
        ],
        compiler_params=pltpu.CompilerParams(needs_layout_passes=False),
    )
    def deg(dst_hbm, out_hbm, didx, cnt):
        c = lax.axis_index("c")
        s = lax.axis_index("s")
        wid = s * NC + c

        def z(i, _):
            cnt[pl.ds(i * 16, 16)] = jnp.zeros((16,), jnp.float32)
            return 0

        lax.fori_loop(0, n_pad // 16, z, 0)
        ones16 = jnp.ones((16,), jnp.float32)
        ebase = wid * ept

        def chunk(k, _):
            base = pl.multiple_of(ebase + k * EB, EB)
            pltpu.sync_copy(dst_hbm.at[pl.ds(base, EB)], didx)

            def inner(j, _):
                idx = didx[pl.ds(j * 16, 16)]
                plsc.addupdate_scatter(cnt, [idx], ones16)
                return 0

            lax.fori_loop(0, EB // 16, inner, 0)
            return 0

        lax.fori_loop(0, ept // EB, chunk, 0)
        pltpu.sync_copy(cnt, out_hbm.at[wid])

    return deg


def _make_agg(n_pad, e_pad, d):
    nw = NC * NS
    ept = e_pad // nw
    n_chunks = ept // CH
    rpt = n_pad // NS
    mesh = plsc.VectorSubcoreMesh(core_axis_name="c", subcore_axis_name="s", num_cores=NC, num_subcores=NS)

    nb = 2
    assert n_chunks % nb == 0

    @functools.partial(
        pl.kernel,
        mesh=mesh,
        out_type=jax.ShapeDtypeStruct((NC, n_pad, d), jnp.float32),
        scratch_types=[
            pltpu.VMEM((nb, CH), jnp.int32),
            pltpu.VMEM((nb, CH), jnp.int32),
            pltpu.VMEM((nb, CH, d), jnp.float32),
            pltpu.VMEM_SHARED((n_pad, d), jnp.float32),
            pltpu.SemaphoreType.DMA,
        ],
    )
    def agg(y_hbm, src_hbm, dst_hbm, out_hbm, sidx, didx, rows, acc, sem):
        c = lax.axis_index("c")
        s = lax.axis_index("s")
        wid = s * NC + c

        def zbuf(i, _):
            rows[0, i // 8, pl.ds((i % 8) * 16, 16)] = jnp.zeros(
                (16,), jnp.float32)
            return 0

        lax.fori_loop(0, CH * d // 16, zbuf, 0)
        row0 = s * rpt

        def zcopy(k, _):
            pltpu.sync_copy(rows.at[0], acc.at[pl.ds(row0 + k * CH, CH), :])
            return 0

        lax.fori_loop(0, rpt // CH, zcopy, 0)
        plsc.subcore_barrier()

        ebase = wid * ept

        def group(g, _):
            gbase = ebase + g * (nb * CH)
            descs = []
            for b in range(nb):
                base = pl.multiple_of(gbase + b * CH, CH)
                pltpu.sync_copy(src_hbm.at[pl.ds(base, CH)], sidx.at[b])
                pltpu.sync_copy(dst_hbm.at[pl.ds(base, CH)], didx.at[b])
                descs.append(
                    pltpu.async_copy(y_hbm.at[sidx.at[b]], rows.at[b], sem))
            for b in range(nb):
                descs[b].wait()
                pltpu.sync_copy(rows.at[b], acc.at[didx.at[b]], add=True)
            return 0

        lax.fori_loop(0, n_chunks // nb, group, 0)
        plsc.subcore_barrier()

        def ocopy(k, _):
            r = pl.multiple_of(row0 + k * CH, CH)
            pltpu.sync_copy(acc.at[pl.ds(r, CH), :], rows.at[0])
            pltpu.sync_copy(rows.at[0], out_hbm.at[c, pl.ds(r, CH), :])
            return 0

        lax.fori_loop(0, rpt // CH, ocopy, 0)

    return agg


def _tc0_body(degp_ref, dinv_ref):
    deg = 1.0 + jnp.sum(degp_ref[...], axis=0)
    dinv_ref[...] = lax.rsqrt(deg)[:, None]


def _tc1_body(x_ref, w_ref, dinv_ref, y_ref):
    xw = jnp.dot(x_ref[...], w_ref[...], preferred_element_type=jnp.float32)
    y_ref[...] = xw * dinv_ref[...]


def _tc2_body(aggp_ref, y1_ref, dinv_ref, b1_ref, t_ref, stats_ref, acc_ref):
    i = pl.program_id(0)
    t = dinv_ref[...] * (aggp_ref[0] + aggp_ref[1] + y1_ref[...]) + b1_ref[...]
    t_ref[...] = t
    ps = jnp.sum(t, axis=0, keepdims=True)
    pq = jnp.sum(t * t, axis=0, keepdims=True)

    @pl.when(i == 0)
    def _():
        acc_ref[0:1, :] = ps
        acc_ref[1:2, :] = pq

    @pl.when(i > 0)
    def _():
        acc_ref[0:1, :] += ps
        acc_ref[1:2, :] += pq

    @pl.when(i == pl.num_programs(0) - 1)
    def _():
        stats_ref[...] = acc_ref[...]


def _tc3_body(n, t_ref, stats_ref, g_ref, be_ref, dinv_ref, w_ref, y2_ref):
    mean = stats_ref[0:1, :] / n
    var = stats_ref[1:2, :] / n - mean * mean
    inv = lax.rsqrt(var + 1e-5)
    h = (t_ref[...] - mean) * inv * g_ref[...] + be_ref[...]
    h = jnp.maximum(h, 0.0)
    y2_ref[...] = dinv_ref[...] * jnp.dot(
        h, w_ref[...], preferred_element_type=jnp.float32)


def _tc4_body(aggp_ref, y2_ref, dinv_ref, b2_ref, out_ref):
    out_ref[...] = (
        dinv_ref[...] * (aggp_ref[0] + aggp_ref[1] + y2_ref[...]) + b2_ref[...])


def _row_specs(r, d):
    return pl.BlockSpec((r, d), lambda i: (i, 0))


def kernel(x, edge_index, W1, b1, gamma, beta, W2, b2):
    n, d_in = x.shape
    d_h = W1.shape[1]
    d_out = W2.shape[1]
    e = edge_index.shape[1]

    grain = NC * NS * EB
    e_pad = -(-e // grain) * grain
    n_pad = -(-n // (NS * CH)) * (NS * CH)

    src = edge_index[0].astype(jnp.int32)
    dst = edge_index[1].astype(jnp.int32)
    src_p = jnp.concatenate([src, jnp.zeros((e_pad - e,), jnp.int32)])
    dst_p = jnp.concatenate([dst, jnp.full((e_pad - e,), n, jnp.int32)])

    degp = _make_deg(n_pad, e_pad)(dst_p)

    r = 2000
    grid = n // r
    f32 = jnp.float32

    dinv = pl.pallas_call(
        _tc0_body,
        in_specs=[pl.BlockSpec((NC * NS, n_pad), lambda: (0, 0))],
        out_specs=pl.BlockSpec((n_pad, 1), lambda: (0, 0)),
        out_shape=jax.ShapeDtypeStruct((n_pad, 1), f32),
    )(degp)

    y1 = pl.pallas_call(
        _tc1_body,
        grid=(grid,),
        in_specs=[
            _row_specs(r, d_in),
            pl.BlockSpec((d_in, d_h), lambda i: (0, 0)),
            pl.BlockSpec((r, 1), lambda i: (i, 0)),
        ],
        out_specs=_row_specs(r, d_h),
        out_shape=jax.ShapeDtypeStruct((n, d_h), f32),
    )(x, W1, dinv)

    agg_fn = _make_agg(n_pad, e_pad, d_h)
    agg1 = agg_fn(y1, src_p, dst_p)

    t, stats = pl.pallas_call(
        _tc2_body,
        grid=(grid,),
        in_specs=[
            pl.BlockSpec((2, r, d_h), lambda i: (0, i, 0)),
            _row_specs(r, d_h),
            pl.BlockSpec((r, 1), lambda i: (i, 0)),
            pl.BlockSpec((d_h,), lambda i: (0,)),
        ],
        out_specs=[_row_specs(r, d_h), pl.BlockSpec((8, d_h), lambda i: (0, 0))],
        out_shape=[
            jax.ShapeDtypeStruct((n, d_h), f32),
            jax.ShapeDtypeStruct((8, d_h), f32),
        ],
        scratch_shapes=[pltpu.VMEM((8, d_h), f32)],
    )(agg1, y1, dinv, b1)

    y2 = pl.pallas_call(
        functools.partial(_tc3_body, float(n)),
        grid=(grid,),
        in_specs=[
            _row_specs(r, d_h),
            pl.BlockSpec((8, d_h), lambda i: (0, 0)),
            pl.BlockSpec((d_h,), lambda i: (0,)),
            pl.BlockSpec((d_h,), lambda i: (0,)),
            pl.BlockSpec((r, 1), lambda i: (i, 0)),
            pl.BlockSpec((d_h, d_out), lambda i: (0, 0)),
        ],
        out_specs=_row_specs(r, d_out),
        out_shape=jax.ShapeDtypeStruct((n, d_out), f32),
    )(t, stats, gamma, beta, dinv, W2)

    agg2 = agg_fn(y2, src_p, dst_p)

    out = pl.pallas_call(
        _tc4_body,
        grid=(grid,),
        in_specs=[
            pl.BlockSpec((2, r, d_out), lambda i: (0, i, 0)),
            _row_specs(r, d_out),
            pl.BlockSpec((r, 1), lambda i: (i, 0)),
            pl.BlockSpec((d_out,), lambda i: (0,)),
        ],
        out_specs=_row_specs(r, d_out),
        out_shape=jax.ShapeDtypeStruct((n, d_out), f32),
    )(agg2, y2, dinv, b2)

    return out


# speedup vs baseline: 9.6847x; 1.1414x over previous
import functools

import jax
import jax.numpy as jnp
from jax import lax
from jax.experimental import pallas as pl
from jax.experimental.pallas import tpu as pltpu
from jax.experimental.pallas import tpu_sc as plsc

NC = 2
NS = 16
CH = 128
EB = 2048


def _make_deg(n_pad, e_pad):
    nw = NC * NS
    ept = e_pad // nw
    assert ept % EB == 0
    mesh = plsc.VectorSubcoreMesh(core_axis_name="c", subcore_axis_name="s",
                                  num_cores=NC, num_subcores=NS)

    @functools.partial(
        pl.kernel,
        mesh=mesh,
        out_type=jax.ShapeDtypeStruct((nw, n_pad), jnp.float32),
        scratch_types=[
            pltpu.VMEM((EB,), jnp.int32),
            pltpu.VMEM((n_pad,), jnp.float32),
        ],
        compiler_params=pltpu.CompilerParams(needs_layout_passes=False),
    )
    def deg(dst_hbm, out_hbm, didx, cnt):
        c = lax.axis_index("c")
        s = lax.axis_index("s")
        wid = s * NC + c

        def z(i, _):
            cnt[pl.ds(i * 16, 16)] = jnp.zeros((16,), jnp.float32)
            return 0

        lax.fori_loop(0, n_pad // 16, z, 0)
        ones16 = jnp.ones((16,), jnp.float32)
        ebase = wid * ept

        def chunk(k, _):
            base = pl.multiple_of(ebase + k * EB, EB)
            pltpu.sync_copy(dst_hbm.at[pl.ds(base, EB)], didx)

            def inner(j, _):
                idx = didx[pl.ds(j * 16, 16)]
                plsc.addupdate_scatter(cnt, [idx], ones16)
                return 0

            lax.fori_loop(0, EB // 16, inner, 0)
            return 0

        lax.fori_loop(0, ept // EB, chunk, 0)
        pltpu.sync_copy(cnt, out_hbm.at[wid])

    return deg


def _make_agg(n_pad, e_pad, d):
    nw = NC * NS
    ept = e_pad // nw
    n_chunks = ept // CH
    rpt = n_pad // NS
    mesh = plsc.VectorSubcoreMesh(core_axis_name="c", subcore_axis_name="s", num_cores=NC, num_subcores=NS)

    nb = 2
    gch = 40
    assert n_chunks % gch == 0 and gch % nb == 0

    @functools.partial(
        pl.kernel,
        mesh=mesh,
        out_type=jax.ShapeDtypeStruct((NC, n_pad, d), jnp.float32),
        scratch_types=[
            pltpu.VMEM((gch, CH), jnp.int32),
            pltpu.VMEM((gch, CH), jnp.int32),
            pltpu.VMEM((nb, CH, d), jnp.float32),
            pltpu.VMEM_SHARED((n_pad, d), jnp.float32),
            pltpu.SemaphoreType.DMA,
        ],
    )
    def agg(y_hbm, src_hbm, dst_hbm, out_hbm, sidx, didx, rows, acc, sem):
        c = lax.axis_index("c")
        s = lax.axis_index("s")
        wid = s * NC + c

        def zbuf(i, _):
            rows[0, i // 8, pl.ds((i % 8) * 16, 16)] = jnp.zeros(
                (16,), jnp.float32)
            return 0

        lax.fori_loop(0, CH * d // 16, zbuf, 0)
        row0 = s * rpt

        def zcopy(k, _):
            pltpu.sync_copy(rows.at[0], acc.at[pl.ds(row0 + k * CH, CH), :])
            return 0

        lax.fori_loop(0, rpt // CH, zcopy, 0)
        plsc.subcore_barrier()

        def group(g, _):
            crow = wid * n_chunks + g * gch
            pltpu.sync_copy(src_hbm.at[pl.ds(crow, gch), :], sidx)
            pltpu.sync_copy(dst_hbm.at[pl.ds(crow, gch), :], didx)

            def pair(p, _):
                descs = []
                for b in range(nb):
                    descs.append(pltpu.async_copy(
                        y_hbm.at[sidx.at[p * nb + b]], rows.at[b], sem))
                for b in range(nb):
                    descs[b].wait()
                    pltpu.sync_copy(
                        rows.at[b], acc.at[didx.at[p * nb + b]], add=True)
                return 0

            lax.fori_loop(0, gch // nb, pair, 0)
            return 0

        lax.fori_loop(0, n_chunks // gch, group, 0)
        plsc.subcore_barrier()

        def ocopy(k, _):
            r = pl.multiple_of(row0 + k * CH, CH)
            pltpu.sync_copy(acc.at[pl.ds(r, CH), :], rows.at[0])
            pltpu.sync_copy(rows.at[0], out_hbm.at[c, pl.ds(r, CH), :])
            return 0

        lax.fori_loop(0, rpt // CH, ocopy, 0)

    return agg


def _tc0_body(degp_ref, dinv_ref):
    deg = 1.0 + jnp.sum(degp_ref[...], axis=0)
    dinv_ref[...] = lax.rsqrt(deg)[:, None]


def _tc1_body(x_ref, w_ref, dinv_ref, y_ref):
    xw = jnp.dot(x_ref[...], w_ref[...], preferred_element_type=jnp.float32)
    y_ref[...] = xw * dinv_ref[...]


def _tc2_body(aggp_ref, y1_ref, dinv_ref, b1_ref, t_ref, stats_ref, acc_ref):
    i = pl.program_id(0)
    t = dinv_ref[...] * (aggp_ref[0] + aggp_ref[1] + y1_ref[...]) + b1_ref[...]
    t_ref[...] = t
    ps = jnp.sum(t, axis=0, keepdims=True)
    pq = jnp.sum(t * t, axis=0, keepdims=True)

    @pl.when(i == 0)
    def _():
        acc_ref[0:1, :] = ps
        acc_ref[1:2, :] = pq

    @pl.when(i > 0)
    def _():
        acc_ref[0:1, :] += ps
        acc_ref[1:2, :] += pq

    @pl.when(i == pl.num_programs(0) - 1)
    def _():
        stats_ref[...] = acc_ref[...]


def _tc3_body(n, t_ref, stats_ref, g_ref, be_ref, dinv_ref, w_ref, y2_ref):
    mean = stats_ref[0:1, :] / n
    var = stats_ref[1:2, :] / n - mean * mean
    inv = lax.rsqrt(var + 1e-5)
    h = (t_ref[...] - mean) * inv * g_ref[...] + be_ref[...]
    h = jnp.maximum(h, 0.0)
    y2_ref[...] = dinv_ref[...] * jnp.dot(
        h, w_ref[...], preferred_element_type=jnp.float32)


def _tc4_body(aggp_ref, y2_ref, dinv_ref, b2_ref, out_ref):
    out_ref[...] = (
        dinv_ref[...] * (aggp_ref[0] + aggp_ref[1] + y2_ref[...]) + b2_ref[...])


def _row_specs(r, d):
    return pl.BlockSpec((r, d), lambda i: (i, 0))


def kernel(x, edge_index, W1, b1, gamma, beta, W2, b2):
    n, d_in = x.shape
    d_h = W1.shape[1]
    d_out = W2.shape[1]
    e = edge_index.shape[1]

    grain = NC * NS * EB
    e_pad = -(-e // grain) * grain
    n_pad = -(-n // (NS * CH)) * (NS * CH)

    src = edge_index[0].astype(jnp.int32)
    dst = edge_index[1].astype(jnp.int32)
    src_p = jnp.concatenate([src, jnp.zeros((e_pad - e,), jnp.int32)])
    dst_p = jnp.concatenate([dst, jnp.full((e_pad - e,), n, jnp.int32)])
    src_p2 = src_p.reshape(e_pad // CH, CH)
    dst_p2 = dst_p.reshape(e_pad // CH, CH)

    degp = _make_deg(n_pad, e_pad)(dst_p)

    r = 2000
    grid = n // r
    f32 = jnp.float32

    dinv = pl.pallas_call(
        _tc0_body,
        in_specs=[pl.BlockSpec((NC * NS, n_pad), lambda: (0, 0))],
        out_specs=pl.BlockSpec((n_pad, 1), lambda: (0, 0)),
        out_shape=jax.ShapeDtypeStruct((n_pad, 1), f32),
    )(degp)

    y1 = pl.pallas_call(
        _tc1_body,
        grid=(grid,),
        in_specs=[
            _row_specs(r, d_in),
            pl.BlockSpec((d_in, d_h), lambda i: (0, 0)),
            pl.BlockSpec((r, 1), lambda i: (i, 0)),
        ],
        out_specs=_row_specs(r, d_h),
        out_shape=jax.ShapeDtypeStruct((n, d_h), f32),
    )(x, W1, dinv)

    agg_fn = _make_agg(n_pad, e_pad, d_h)
    agg1 = agg_fn(y1, src_p2, dst_p2)

    t, stats = pl.pallas_call(
        _tc2_body,
        grid=(grid,),
        in_specs=[
            pl.BlockSpec((2, r, d_h), lambda i: (0, i, 0)),
            _row_specs(r, d_h),
            pl.BlockSpec((r, 1), lambda i: (i, 0)),
            pl.BlockSpec((d_h,), lambda i: (0,)),
        ],
        out_specs=[_row_specs(r, d_h), pl.BlockSpec((8, d_h), lambda i: (0, 0))],
        out_shape=[
            jax.ShapeDtypeStruct((n, d_h), f32),
            jax.ShapeDtypeStruct((8, d_h), f32),
        ],
        scratch_shapes=[pltpu.VMEM((8, d_h), f32)],
    )(agg1, y1, dinv, b1)

    y2 = pl.pallas_call(
        functools.partial(_tc3_body, float(n)),
        grid=(grid,),
        in_specs=[
            _row_specs(r, d_h),
            pl.BlockSpec((8, d_h), lambda i: (0, 0)),
            pl.BlockSpec((d_h,), lambda i: (0,)),
            pl.BlockSpec((d_h,), lambda i: (0,)),
            pl.BlockSpec((r, 1), lambda i: (i, 0)),
            pl.BlockSpec((d_h, d_out), lambda i: (0, 0)),
        ],
        out_specs=_row_specs(r, d_out),
        out_shape=jax.ShapeDtypeStruct((n, d_out), f32),
    )(t, stats, gamma, beta, dinv, W2)

    agg2 = agg_fn(y2, src_p2, dst_p2)

    out = pl.pallas_call(
        _tc4_body,
        grid=(grid,),
        in_specs=[
            pl.BlockSpec((2, r, d_out), lambda i: (0, i, 0)),
            _row_specs(r, d_out),
            pl.BlockSpec((r, 1), lambda i: (i, 0)),
            pl.BlockSpec((d_out,), lambda i: (0,)),
        ],
        out_specs=_row_specs(r, d_out),
        out_shape=jax.ShapeDtypeStruct((n, d_out), f32),
    )(agg2, y2, dinv, b2)

    return out
